# baseline (speedup 1.0000x reference)
import functools

import jax
import jax.numpy as jnp
from jax import lax
from jax.experimental import pallas as pl
from jax.experimental.pallas import tpu as pltpu
from jax.experimental.pallas import tpu_sc as plsc

_B = 128
_V = 100000
_OUT = 2 * _V - 1

_NC = 2
_NS = 16
_NW = _NC * _NS
_SLAB = _V // _NW
_CH = 125
_NCH = _SLAB // _CH

_RB_OUT = 16000
_NB_OUT = (_OUT + _RB_OUT - 1) // _RB_OUT

_BIG = 2**30


def _sc_argmax_body(x0_hbm, x1_hbm, pm0_hbm, pi0_hbm, pm1_hbm, pi1_hbm,
                    buf0, buf1, buf2, buf3, facc, iacc,
                    sem0, sem1, sem2, sem3):
    c = lax.axis_index("c")
    s = lax.axis_index("s")
    w = s * _NC + c
    v0 = w * _SLAB
    bufs = (buf0, buf1, buf2, buf3)
    sems = (sem0, sem1, sem2, sem3)
    nbuf = 4

    for x_hbm, pm_hbm, pi_hbm in ((x0_hbm, pm0_hbm, pi0_hbm),
                                  (x1_hbm, pm1_hbm, pi1_hbm)):
        for p in range(nbuf - 1):
            pltpu.make_async_copy(
                x_hbm.at[pl.ds((v0 + p * _CH) * _B, _CH * _B)],
                bufs[p], sems[p],
            ).start()

        carry = tuple(jnp.full((16,), -1.0, jnp.float32) for _ in range(8)) + \
                tuple(jnp.zeros((16,), jnp.int32) for _ in range(8))

        for ci in range(_NCH):
            buf = bufs[ci % nbuf]
            nxt = ci + nbuf - 1
            if nxt < _NCH:
                pltpu.make_async_copy(
                    x_hbm.at[pl.ds((v0 + nxt * _CH) * _B, _CH * _B)],
                    bufs[nxt % nbuf], sems[nxt % nbuf],
                ).start()
            pltpu.make_async_copy(
                x_hbm.at[pl.ds((v0 + ci * _CH) * _B, _CH * _B)],
                buf, sems[ci % nbuf],
            ).wait()
            chunk_v = v0 + ci * _CH

            def row_body(r, cy, buf=buf, chunk_v=chunk_v):
                bs = list(cy[:8])
                bi = list(cy[8:])
                base = r * _B
                jv = jnp.full((16,), chunk_v + r, jnp.int32)
                for k in range(8):
                    v = buf[pl.ds(base + k * 16, 16)]
                    u = v > bs[k]
                    bs[k] = jnp.where(u, v, bs[k])
                    bi[k] = jnp.where(u, jv, bi[k])
                return (*bs, *bi)

            carry = lax.fori_loop(0, _CH, row_body, carry, unroll=5)

        for k in range(8):
            facc[pl.ds(k * 16, 16)] = carry[k]
            iacc[pl.ds(k * 16, 16)] = carry[8 + k]
        pltpu.sync_copy(facc, pm_hbm.at[pl.ds(w * _B, _B)])
        pltpu.sync_copy(iacc, pi_hbm.at[pl.ds(w * _B, _B)])


def _sc_argmax(x0f, x1f):
    kern = functools.partial(
        pl.kernel,
        mesh=plsc.VectorSubcoreMesh(core_axis_name="c", subcore_axis_name="s"),
        out_type=[
            jax.ShapeDtypeStruct((_NW * _B,), jnp.float32),
            jax.ShapeDtypeStruct((_NW * _B,), jnp.int32),
            jax.ShapeDtypeStruct((_NW * _B,), jnp.float32),
            jax.ShapeDtypeStruct((_NW * _B,), jnp.int32),
        ],
        scratch_types=[
            pltpu.VMEM((_CH * _B,), jnp.float32),
            pltpu.VMEM((_CH * _B,), jnp.float32),
            pltpu.VMEM((_CH * _B,), jnp.float32),
            pltpu.VMEM((_CH * _B,), jnp.float32),
            pltpu.VMEM((_B,), jnp.float32),
            pltpu.VMEM((_B,), jnp.int32),
            pltpu.SemaphoreType.DMA,
            pltpu.SemaphoreType.DMA,
            pltpu.SemaphoreType.DMA,
            pltpu.SemaphoreType.DMA,
        ],
    )(_sc_argmax_body)
    return kern(x0f, x1f)


def _onehot_body(pm0, pi0, pm1, pi1, out_ref, res_scr):
    j = pl.program_id(0)

    @pl.when(j == 0)
    def _merge():
        res = jnp.zeros((1, _B), jnp.int32)
        for pm, pi in ((pm0, pi0), (pm1, pi1)):
            mx = jnp.max(pm[...], axis=0, keepdims=True)
            cand = jnp.where(pm[...] == mx, pi[...], jnp.int32(_BIG))
            res = res + jnp.min(cand, axis=0, keepdims=True)
        res_scr[...] = res

    row = jax.lax.broadcasted_iota(jnp.int32, (_RB_OUT, _B), 0) + j * _RB_OUT
    out_ref[...] = (row == res_scr[...]).astype(jnp.float32)


def kernel(x0, x1):
    x0f = x0.T.reshape(-1)
    x1f = x1.T.reshape(-1)
    pm0, pi0, pm1, pi1 = _sc_argmax(x0f, x1f)
    pm0 = pm0.reshape(_NW, _B)
    pi0 = pi0.reshape(_NW, _B)
    pm1 = pm1.reshape(_NW, _B)
    pi1 = pi1.reshape(_NW, _B)
    out_t = pl.pallas_call(
        _onehot_body,
        grid=(_NB_OUT,),
        in_specs=[
            pl.BlockSpec((_NW, _B), lambda j: (0, 0)),
            pl.BlockSpec((_NW, _B), lambda j: (0, 0)),
            pl.BlockSpec((_NW, _B), lambda j: (0, 0)),
            pl.BlockSpec((_NW, _B), lambda j: (0, 0)),
        ],
        out_specs=pl.BlockSpec((_RB_OUT, _B), lambda j: (j, 0)),
        out_shape=jax.ShapeDtypeStruct((_OUT, _B), jnp.float32),
        scratch_shapes=[pltpu.VMEM((1, _B), jnp.int32)],
    )(pm0, pi0, pm1, pi1)
    return out_t.T


# speedup vs baseline: 1.7990x; 1.7990x over previous
import jax
import jax.numpy as jnp
from jax import lax
from jax.experimental import pallas as pl
from jax.experimental.pallas import tpu as pltpu

_B = 128
_V = 100000
_OUT = 2 * _V - 1

_G = 16
_RB_IN = 10000
_NB_IN = _V // _RB_IN
_NG = _RB_IN // _G
_RB_OUT = 16000
_NB_OUT = (_OUT + _RB_OUT - 1) // _RB_OUT

_BIG = 2**30


def _fused_body(x0_ref, x1_ref, out_ref, m0, i0, m1, i1, res_scr):
    i = pl.program_id(0)

    @pl.when(i == 0)
    def _init():
        m0[...] = jnp.full_like(m0, -1.0)
        i0[...] = jnp.zeros_like(i0)
        m1[...] = jnp.full_like(m1, -1.0)
        i1[...] = jnp.zeros_like(i1)

    @pl.when(i < _NB_IN)
    def _argmax_phase():
        base_g = i * _NG

        def body(j, carry):
            ma, ia, mb, ib = carry
            va = x0_ref[pl.ds(j * _G, _G), :]
            vb = x1_ref[pl.ds(j * _G, _G), :]
            jv = jnp.full((_G, _B), base_g + j, jnp.int32)
            ua = va > ma
            ub = vb > mb
            ma = jnp.where(ua, va, ma)
            ia = jnp.where(ua, jv, ia)
            mb = jnp.where(ub, vb, mb)
            ib = jnp.where(ub, jv, ib)
            return ma, ia, mb, ib

        ma, ia, mb, ib = lax.fori_loop(
            0, _NG, body,
            (m0[...], i0[...], m1[...], i1[...]), unroll=4,
        )
        m0[...] = ma
        i0[...] = ia
        m1[...] = mb
        i1[...] = ib

        @pl.when(i == _NB_IN - 1)
        def _collapse():
            r = jax.lax.broadcasted_iota(jnp.int32, (_G, _B), 0)
            res = jnp.zeros((1, _B), jnp.int32)
            for macc, vidx in ((m0[...], i0[...]), (m1[...], i1[...])):
                mx = jnp.max(macc, axis=0, keepdims=True)
                rows = vidx * _G + r
                cand = jnp.where(macc == mx, rows, jnp.int32(_BIG))
                res = res + jnp.min(cand, axis=0, keepdims=True)
            res_scr[...] = res

    @pl.when(i >= _NB_IN)
    def _onehot_phase():
        j = i - _NB_IN
        res = res_scr[...]
        row = jax.lax.broadcasted_iota(jnp.int32, (_RB_OUT, _B), 0) + j * _RB_OUT
        out_ref[...] = (row == res).astype(jnp.float32)


def kernel(x0, x1):
    out_t = pl.pallas_call(
        _fused_body,
        grid=(_NB_IN + _NB_OUT,),
        in_specs=[
            pl.BlockSpec((_RB_IN, _B), lambda i: (jnp.minimum(i, _NB_IN - 1), 0)),
            pl.BlockSpec((_RB_IN, _B), lambda i: (jnp.minimum(i, _NB_IN - 1), 0)),
        ],
        out_specs=pl.BlockSpec(
            (_RB_OUT, _B), lambda i: (jnp.maximum(i - _NB_IN, 0), 0)
        ),
        out_shape=jax.ShapeDtypeStruct((_OUT, _B), jnp.float32),
        scratch_shapes=[
            pltpu.VMEM((_G, _B), jnp.float32),
            pltpu.VMEM((_G, _B), jnp.int32),
            pltpu.VMEM((_G, _B), jnp.float32),
            pltpu.VMEM((_G, _B), jnp.int32),
            pltpu.VMEM((1, _B), jnp.int32),
        ],
    )(x0.T, x1.T)
    return out_t.T
